# baseline (speedup 1.0000x reference)
import functools

import jax
import jax.numpy as jnp
from jax import lax
from jax.experimental import pallas as pl
from jax.experimental.pallas import tpu as pltpu, tpu_sc as plsc

B, L, D, H, DFF, N = 16, 512, 1024, 8, 4096, 4096
DH = D // H

NC, NS = 2, 16
NW = NC * NS


def _sc_gather(table, idx, n_rows, chunk):
  per_w = n_rows // NW
  n_chunks = per_w // chunk
  mesh = plsc.VectorSubcoreMesh(core_axis_name="c", subcore_axis_name="s")

  @functools.partial(
      pl.kernel,
      mesh=mesh,
      out_type=jax.ShapeDtypeStruct((n_rows, D), jnp.float32),
      scratch_types=[
          pltpu.VMEM((n_chunks, chunk), jnp.int32),
          pltpu.VMEM((chunk, D), jnp.float32),
          pltpu.SemaphoreType.DMA,
      ],
  )
  def gather_kernel(idx_hbm, table_hbm, out_hbm, idx_v, rows_v, sem):
    wid = lax.axis_index("s") * NC + lax.axis_index("c")
    base = wid * per_w
    pltpu.sync_copy(idx_hbm.at[wid], idx_v)
    for c in range(n_chunks):
      pltpu.async_copy(table_hbm.at[idx_v.at[c]], rows_v, sem).wait()
      pltpu.sync_copy(rows_v, out_hbm.at[pl.ds(base + c * chunk, chunk)])

  return gather_kernel(idx.reshape(NW, n_chunks, chunk), table)


def _ln_rows(y, g, b):
  m = jnp.mean(y, axis=1, keepdims=True)
  d = y - m
  v = jnp.mean(d * d, axis=1, keepdims=True)
  return d * lax.rsqrt(v + 1e-5) * g + b


def _attn_body(len_ref, x_ref, wq_ref, wk_ref, wv_ref, wo_ref,
               bq_ref, bk_ref, bv_ref, bo_ref,
               g1_ref, t1_ref, out_ref, ctx_ref):
  bidx = pl.program_id(0)
  seqlen = len_ref[bidx, 0]
  cols = lax.broadcasted_iota(jnp.int32, (1, L), 1)
  addmask = jnp.where(cols < seqlen, 0.0, -1e9)
  rows = lax.broadcasted_iota(jnp.int32, (L, 1), 0)
  vrow = (rows < seqlen).astype(jnp.float32)

  x = x_ref[0] * vrow
  q = jnp.dot(x, wq_ref[...], preferred_element_type=jnp.float32) + bq_ref[...]
  k = jnp.dot(x, wk_ref[...], preferred_element_type=jnp.float32) + bk_ref[...]
  v = jnp.dot(x, wv_ref[...], preferred_element_type=jnp.float32) + bv_ref[...]
  scale = 1.0 / (DH ** 0.5)
  for h in range(H):
    sl = slice(h * DH, (h + 1) * DH)
    qh, kh, vh = q[:, sl], k[:, sl], v[:, sl]
    s = lax.dot_general(qh, kh, (((1,), (1,)), ((), ())),
                        preferred_element_type=jnp.float32)
    s = s * scale + addmask
    m = jnp.max(s, axis=1, keepdims=True)
    e = jnp.exp(s - m)
    p = e / jnp.sum(e, axis=1, keepdims=True)
    ctx_ref[:, sl] = jnp.dot(p, vh, preferred_element_type=jnp.float32)
  att = jnp.dot(ctx_ref[...], wo_ref[...],
                preferred_element_type=jnp.float32) + bo_ref[...]
  out_ref[0] = _ln_rows(x + att, g1_ref[...], t1_ref[...])


def _attention(x, lengths, Wq, Wk, Wv, Wo, bq, bk, bv, bo, ln1_g, ln1_b):
  full = lambda shape: pl.BlockSpec(shape, lambda b: (0,) * len(shape))
  nb = x.shape[0]
  return pl.pallas_call(
      _attn_body,
      grid=(nb,),
      in_specs=[
          pl.BlockSpec(memory_space=pltpu.SMEM),
          pl.BlockSpec((1, L, D), lambda b: (b, 0, 0)),
          full((D, D)), full((D, D)), full((D, D)), full((D, D)),
          full((1, D)), full((1, D)), full((1, D)), full((1, D)),
          full((1, D)), full((1, D)),
      ],
      out_specs=pl.BlockSpec((1, L, D), lambda b: (b, 0, 0)),
      out_shape=jax.ShapeDtypeStruct((nb, L, D), jnp.float32),
      scratch_shapes=[pltpu.VMEM((L, D), jnp.float32)],
  )(lengths, x, Wq, Wk, Wv, Wo, bq, bk, bv, bo, ln1_g, ln1_b)


_FR = 128
_FSTEPS = (B * L) // _FR
_WNB = N // _FSTEPS
_WPC = 1024


def _ffn_body(len_ref, h_ref, w1_ref, w2_ref, b1_ref, b2_ref, g2_ref, t2_ref,
              out_ref):
  i = pl.program_id(0)
  nsub = L // _FR
  e = i // nsub
  j = i % nsub

  @pl.when(j * _FR < len_ref[e, 0])
  def _():
    h1 = h_ref[...]
    mid = jnp.maximum(
        jnp.dot(h1, w1_ref[...], preferred_element_type=jnp.float32)
        + b1_ref[...], 0.0)
    ff = (jnp.dot(mid, w2_ref[...], preferred_element_type=jnp.float32)
          + b2_ref[...])
    out_ref[...] = _ln_rows(h1 + ff, g2_ref[...], t2_ref[...])


def _ffn(h1, lengths, W1, W2, b1, b2, ln2_g, ln2_b):
  full = lambda shape: pl.BlockSpec(shape, lambda i: (0,) * len(shape))
  rows = B * L
  return pl.pallas_call(
      _ffn_body,
      grid=(rows // _FR,),
      in_specs=[
          pl.BlockSpec(memory_space=pltpu.SMEM),
          pl.BlockSpec((_FR, D), lambda i: (i, 0)),
          full((D, DFF)), full((DFF, D)),
          full((1, DFF)), full((1, D)), full((1, D)), full((1, D)),
      ],
      out_specs=pl.BlockSpec((_FR, D), lambda i: (i, 0)),
      out_shape=jax.ShapeDtypeStruct((rows, D), jnp.float32),
  )(lengths, h1, W1, W2, b1, b2, ln2_g, ln2_b)


_NB = 8
_NBS = N // _NB
_PBS = 1024


def _winner_body(idx_ref, widx_ref, valid_ref):
  for nb in range(_NB):
    nids = nb * _NBS + lax.broadcasted_iota(jnp.int32, (1, _NBS), 1)
    best = jnp.full((1, _NBS), -1, jnp.int32)
    for pb in range(B * L // _PBS):
      c = idx_ref[pl.ds(pb * _PBS, _PBS), :]
      pos = pb * _PBS + lax.broadcasted_iota(jnp.int32, (_PBS, _NBS), 0)
      cand = jnp.where(c == nids, pos, -1)
      best = jnp.maximum(best, jnp.max(cand, axis=0, keepdims=True))
    widx_ref[pl.ds(nb, 1), :] = jnp.maximum(best, 0)
    valid_ref[pl.ds(nb, 1), :] = (best >= 0).astype(jnp.float32)


def _winner(idx_flat):
  return pl.pallas_call(
      _winner_body,
      in_specs=[pl.BlockSpec((B * L, 1), lambda: (0, 0))],
      out_specs=[pl.BlockSpec((_NB, _NBS), lambda: (0, 0)),
                 pl.BlockSpec((_NB, _NBS), lambda: (0, 0))],
      out_shape=[jax.ShapeDtypeStruct((_NB, _NBS), jnp.int32),
                 jax.ShapeDtypeStruct((_NB, _NBS), jnp.float32)],
  )(idx_flat.reshape(B * L, 1))


def _final_ln_body(y_ref, valid_ref, g_ref, b_ref, out_ref):
  y = y_ref[...] * valid_ref[...]
  out_ref[...] = _ln_rows(y, g_ref[...], b_ref[...])


def _final_ln(rows, valid, norm_g, norm_b):
  blk = 512
  return pl.pallas_call(
      _final_ln_body,
      grid=(N // blk,),
      in_specs=[
          pl.BlockSpec((blk, D), lambda i: (i, 0)),
          pl.BlockSpec((blk, 1), lambda i: (i, 0)),
          pl.BlockSpec((1, D), lambda i: (0, 0)),
          pl.BlockSpec((1, D), lambda i: (0, 0)),
      ],
      out_specs=pl.BlockSpec((blk, D), lambda i: (i, 0)),
      out_shape=jax.ShapeDtypeStruct((N, D), jnp.float32),
  )(rows, valid, norm_g, norm_b)


def kernel(cfg_nodes_encodings, permutations, lengths, Wq, bq, Wk, bk, Wv, bv,
           Wo, bo, W1, b1, W2, b2, ln1_g, ln1_b, ln2_g, ln2_b, norm_g, norm_b):
  mask0 = jnp.arange(L, dtype=jnp.int32)[None, :] < lengths[:, None]
  perm_flat = jnp.where(mask0, permutations, 0).reshape(B * L)
  x_flat = _sc_gather(cfg_nodes_encodings, perm_flat, B * L, 64)

  mask = jnp.arange(L, dtype=jnp.int32)[None, :] < lengths[:, None]
  idx_flat = jnp.where(mask, permutations, N).reshape(B * L)
  widx, valid = _winner(idx_flat)

  lens2 = lengths.reshape(B, 1)
  h1 = _attention(x_flat.reshape(B, L, D), lens2, Wq, Wk, Wv, Wo,
                  bq.reshape(1, D), bk.reshape(1, D), bv.reshape(1, D),
                  bo.reshape(1, D), ln1_g.reshape(1, D), ln1_b.reshape(1, D))

  h = _ffn(h1.reshape(B * L, D), lens2, W1, W2,
           b1.reshape(1, DFF), b2.reshape(1, D),
           ln2_g.reshape(1, D), ln2_b.reshape(1, D))

  rows = _sc_gather(h, widx.reshape(N), N, 64)
  return _final_ln(rows, valid.reshape(N, 1),
                   norm_g.reshape(1, D), norm_b.reshape(1, D))


# speedup vs baseline: 1.3414x; 1.3414x over previous
import functools

import jax
import jax.numpy as jnp
from jax import lax
from jax.experimental import pallas as pl
from jax.experimental.pallas import tpu as pltpu, tpu_sc as plsc

B, L, D, H, DFF, N = 16, 512, 1024, 8, 4096, 4096
DH = D // H

NC, NS = 2, 16
NW = NC * NS


def _sc_gather(table, idx, n_rows, chunk):
  per_w = n_rows // NW
  n_chunks = per_w // chunk
  mesh = plsc.VectorSubcoreMesh(core_axis_name="c", subcore_axis_name="s")

  @functools.partial(
      pl.kernel,
      mesh=mesh,
      out_type=jax.ShapeDtypeStruct((n_rows, D), jnp.float32),
      scratch_types=[
          pltpu.VMEM((n_chunks, chunk), jnp.int32),
          pltpu.VMEM((chunk, D), jnp.float32),
          pltpu.SemaphoreType.DMA,
      ],
  )
  def gather_kernel(idx_hbm, table_hbm, out_hbm, idx_v, rows_v, sem):
    wid = lax.axis_index("s") * NC + lax.axis_index("c")
    base = wid * per_w
    pltpu.sync_copy(idx_hbm.at[wid], idx_v)
    for c in range(n_chunks):
      pltpu.async_copy(table_hbm.at[idx_v.at[c]], rows_v, sem).wait()
      pltpu.sync_copy(rows_v, out_hbm.at[pl.ds(base + c * chunk, chunk)])

  return gather_kernel(idx.reshape(NW, n_chunks, chunk), table)


def _ln_rows(y, g, b):
  m = jnp.mean(y, axis=1, keepdims=True)
  d = y - m
  v = jnp.mean(d * d, axis=1, keepdims=True)
  return d * lax.rsqrt(v + 1e-5) * g + b


_AR = 128
_ANB = L // _AR


def _attn_body(len_ref, x_ref, wq_ref, wk_ref, wv_ref, wo_ref,
               bq_ref, bk_ref, bv_ref, bo_ref,
               g1_ref, t1_ref, out_ref, k_scr, v_scr, ctx_scr):
  bidx = pl.program_id(0)
  seqlen = len_ref[bidx, 0]
  cols = lax.broadcasted_iota(jnp.int32, (1, L), 1)
  addmask = jnp.where(cols < seqlen, 0.0, -1e9)
  scale = 1.0 / (DH ** 0.5)
  f32 = jnp.float32

  for r in range(_ANB):
    rs = pl.ds(r * _AR, _AR)
    active = r * _AR < seqlen

    @pl.when(active)
    def _():
      xr = x_ref[0, rs, :]
      k_scr[rs, :] = jnp.dot(xr, wk_ref[...], preferred_element_type=f32) + bk_ref[...]
      v_scr[rs, :] = jnp.dot(xr, wv_ref[...], preferred_element_type=f32) + bv_ref[...]

    @pl.when(jnp.logical_not(active))
    def _():
      k_scr[rs, :] = jnp.zeros((_AR, D), f32)
      v_scr[rs, :] = jnp.zeros((_AR, D), f32)

  for r in range(_ANB):
    rs = pl.ds(r * _AR, _AR)

    @pl.when(r * _AR < seqlen)
    def _():
      xr = x_ref[0, rs, :]
      qr = jnp.dot(xr, wq_ref[...], preferred_element_type=f32) + bq_ref[...]
      for h in range(H):
        sl = slice(h * DH, (h + 1) * DH)
        s = lax.dot_general(qr[:, sl], k_scr[:, sl], (((1,), (1,)), ((), ())),
                            preferred_element_type=f32)
        s = s * scale + addmask
        m = jnp.max(s, axis=1, keepdims=True)
        e = jnp.exp(s - m)
        p = e / jnp.sum(e, axis=1, keepdims=True)
        ctx_scr[:, sl] = jnp.dot(p, v_scr[:, sl], preferred_element_type=f32)
      att = jnp.dot(ctx_scr[...], wo_ref[...],
                    preferred_element_type=f32) + bo_ref[...]
      out_ref[0, rs, :] = _ln_rows(xr + att, g1_ref[...], t1_ref[...])


def _attention(x, lengths, Wq, Wk, Wv, Wo, bq, bk, bv, bo, ln1_g, ln1_b):
  full = lambda shape: pl.BlockSpec(shape, lambda b: (0,) * len(shape))
  nb = x.shape[0]
  return pl.pallas_call(
      _attn_body,
      grid=(nb,),
      in_specs=[
          pl.BlockSpec(memory_space=pltpu.SMEM),
          pl.BlockSpec((1, L, D), lambda b: (b, 0, 0)),
          full((D, D)), full((D, D)), full((D, D)), full((D, D)),
          full((1, D)), full((1, D)), full((1, D)), full((1, D)),
          full((1, D)), full((1, D)),
      ],
      out_specs=pl.BlockSpec((1, L, D), lambda b: (b, 0, 0)),
      out_shape=jax.ShapeDtypeStruct((nb, L, D), jnp.float32),
      scratch_shapes=[pltpu.VMEM((L, D), jnp.float32),
                      pltpu.VMEM((L, D), jnp.float32),
                      pltpu.VMEM((_AR, D), jnp.float32)],
  )(lengths, x, Wq, Wk, Wv, Wo, bq, bk, bv, bo, ln1_g, ln1_b)


_FR = 128
_FSTEPS = (B * L) // _FR
_WNB = N // _FSTEPS
_WPC = 1024


def _ffn_body(len_ref, h_ref, w1_ref, w2_ref, b1_ref, b2_ref, g2_ref, t2_ref,
              out_ref):
  i = pl.program_id(0)
  nsub = L // _FR
  e = i // nsub
  j = i % nsub

  @pl.when(j * _FR < len_ref[e, 0])
  def _():
    h1 = h_ref[...]
    mid = jnp.maximum(
        jnp.dot(h1, w1_ref[...], preferred_element_type=jnp.float32)
        + b1_ref[...], 0.0)
    ff = (jnp.dot(mid, w2_ref[...], preferred_element_type=jnp.float32)
          + b2_ref[...])
    out_ref[...] = _ln_rows(h1 + ff, g2_ref[...], t2_ref[...])


def _ffn(h1, lengths, W1, W2, b1, b2, ln2_g, ln2_b):
  full = lambda shape: pl.BlockSpec(shape, lambda i: (0,) * len(shape))
  rows = B * L
  return pl.pallas_call(
      _ffn_body,
      grid=(rows // _FR,),
      in_specs=[
          pl.BlockSpec(memory_space=pltpu.SMEM),
          pl.BlockSpec((_FR, D), lambda i: (i, 0)),
          full((D, DFF)), full((DFF, D)),
          full((1, DFF)), full((1, D)), full((1, D)), full((1, D)),
      ],
      out_specs=pl.BlockSpec((_FR, D), lambda i: (i, 0)),
      out_shape=jax.ShapeDtypeStruct((rows, D), jnp.float32),
  )(lengths, h1, W1, W2, b1, b2, ln2_g, ln2_b)


_NB = 8
_NBS = N // _NB
_PBS = 1024


def _winner_body(idx_ref, widx_ref, valid_ref):
  for nb in range(_NB):
    nids = nb * _NBS + lax.broadcasted_iota(jnp.int32, (1, _NBS), 1)
    best = jnp.full((1, _NBS), -1, jnp.int32)
    for pb in range(B * L // _PBS):
      c = idx_ref[pl.ds(pb * _PBS, _PBS), :]
      pos = pb * _PBS + lax.broadcasted_iota(jnp.int32, (_PBS, _NBS), 0)
      cand = jnp.where(c == nids, pos, -1)
      best = jnp.maximum(best, jnp.max(cand, axis=0, keepdims=True))
    widx_ref[pl.ds(nb, 1), :] = jnp.maximum(best, 0)
    valid_ref[pl.ds(nb, 1), :] = (best >= 0).astype(jnp.float32)


def _winner(idx_flat):
  return pl.pallas_call(
      _winner_body,
      in_specs=[pl.BlockSpec((B * L, 1), lambda: (0, 0))],
      out_specs=[pl.BlockSpec((_NB, _NBS), lambda: (0, 0)),
                 pl.BlockSpec((_NB, _NBS), lambda: (0, 0))],
      out_shape=[jax.ShapeDtypeStruct((_NB, _NBS), jnp.int32),
                 jax.ShapeDtypeStruct((_NB, _NBS), jnp.float32)],
  )(idx_flat.reshape(B * L, 1))


def _final_ln_body(y_ref, valid_ref, g_ref, b_ref, out_ref):
  y = y_ref[...] * valid_ref[...]
  out_ref[...] = _ln_rows(y, g_ref[...], b_ref[...])


def _final_ln(rows, valid, norm_g, norm_b):
  blk = 512
  return pl.pallas_call(
      _final_ln_body,
      grid=(N // blk,),
      in_specs=[
          pl.BlockSpec((blk, D), lambda i: (i, 0)),
          pl.BlockSpec((blk, 1), lambda i: (i, 0)),
          pl.BlockSpec((1, D), lambda i: (0, 0)),
          pl.BlockSpec((1, D), lambda i: (0, 0)),
      ],
      out_specs=pl.BlockSpec((blk, D), lambda i: (i, 0)),
      out_shape=jax.ShapeDtypeStruct((N, D), jnp.float32),
  )(rows, valid, norm_g, norm_b)


def kernel(cfg_nodes_encodings, permutations, lengths, Wq, bq, Wk, bk, Wv, bv,
           Wo, bo, W1, b1, W2, b2, ln1_g, ln1_b, ln2_g, ln2_b, norm_g, norm_b):
  perm_flat = permutations.reshape(B * L)
  x_flat = _sc_gather(cfg_nodes_encodings, perm_flat, B * L, 64)

  mask = jnp.arange(L, dtype=jnp.int32)[None, :] < lengths[:, None]
  idx_flat = jnp.where(mask, permutations, N).reshape(B * L)
  widx, valid = _winner(idx_flat)

  lens2 = lengths.reshape(B, 1)
  h1 = _attention(x_flat.reshape(B, L, D), lens2, Wq, Wk, Wv, Wo,
                  bq.reshape(1, D), bk.reshape(1, D), bv.reshape(1, D),
                  bo.reshape(1, D), ln1_g.reshape(1, D), ln1_b.reshape(1, D))

  h = _ffn(h1.reshape(B * L, D), lens2, W1, W2,
           b1.reshape(1, DFF), b2.reshape(1, D),
           ln2_g.reshape(1, D), ln2_b.reshape(1, D))

  rows = _sc_gather(h, widx.reshape(N), N, 64)
  return _final_ln(rows, valid.reshape(N, 1),
                   norm_g.reshape(1, D), norm_b.reshape(1, D))


# speedup vs baseline: 1.5016x; 1.1194x over previous
import functools

import jax
import jax.numpy as jnp
from jax import lax
from jax.experimental import pallas as pl
from jax.experimental.pallas import tpu as pltpu, tpu_sc as plsc

B, L, D, H, DFF, N = 16, 512, 1024, 8, 4096, 4096
DH = D // H

NC, NS = 2, 16
NW = NC * NS


def _sc_gather(table, idx, n_rows, chunk):
  per_w = n_rows // NW
  n_chunks = per_w // chunk
  mesh = plsc.VectorSubcoreMesh(core_axis_name="c", subcore_axis_name="s")

  @functools.partial(
      pl.kernel,
      mesh=mesh,
      out_type=jax.ShapeDtypeStruct((n_rows, D), jnp.float32),
      scratch_types=[
          pltpu.VMEM((n_chunks, chunk), jnp.int32),
          pltpu.VMEM((chunk, D), jnp.float32),
          pltpu.SemaphoreType.DMA,
      ],
  )
  def gather_kernel(idx_hbm, table_hbm, out_hbm, idx_v, rows_v, sem):
    wid = lax.axis_index("s") * NC + lax.axis_index("c")
    base = wid * per_w
    pltpu.sync_copy(idx_hbm.at[wid], idx_v)
    for c in range(n_chunks):
      pltpu.async_copy(table_hbm.at[idx_v.at[c]], rows_v, sem).wait()
      pltpu.sync_copy(rows_v, out_hbm.at[pl.ds(base + c * chunk, chunk)])

  return gather_kernel(idx.reshape(NW, n_chunks, chunk), table)


def _ln_rows(y, g, b):
  m = jnp.mean(y, axis=1, keepdims=True)
  d = y - m
  v = jnp.mean(d * d, axis=1, keepdims=True)
  return d * lax.rsqrt(v + 1e-5) * g + b


def _attn_body(len_ref, x_ref, wq_ref, wk_ref, wv_ref, wo_ref,
               bq_ref, bk_ref, bv_ref, bo_ref,
               g1_ref, t1_ref, out_ref, ctx_ref):
  bidx = pl.program_id(0)
  seqlen = len_ref[bidx, 0]
  cols = lax.broadcasted_iota(jnp.int32, (1, L), 1)
  addmask = jnp.where(cols < seqlen, 0.0, -1e9)
  x = x_ref[0]
  q = jnp.dot(x, wq_ref[...], preferred_element_type=jnp.float32) + bq_ref[...]
  k = jnp.dot(x, wk_ref[...], preferred_element_type=jnp.float32) + bk_ref[...]
  v = jnp.dot(x, wv_ref[...], preferred_element_type=jnp.float32) + bv_ref[...]
  scale = 1.0 / (DH ** 0.5)
  for h in range(H):
    sl = slice(h * DH, (h + 1) * DH)
    qh, kh, vh = q[:, sl], k[:, sl], v[:, sl]
    s = lax.dot_general(qh, kh, (((1,), (1,)), ((), ())),
                        preferred_element_type=jnp.float32)
    s = s * scale + addmask
    m = jnp.max(s, axis=1, keepdims=True)
    e = jnp.exp(s - m)
    p = e / jnp.sum(e, axis=1, keepdims=True)
    ctx_ref[:, sl] = jnp.dot(p, vh, preferred_element_type=jnp.float32)
  att = jnp.dot(ctx_ref[...], wo_ref[...],
                preferred_element_type=jnp.float32) + bo_ref[...]
  out_ref[0] = _ln_rows(x + att, g1_ref[...], t1_ref[...])


def _attention(x, lengths, Wq, Wk, Wv, Wo, bq, bk, bv, bo, ln1_g, ln1_b):
  full = lambda shape: pl.BlockSpec(shape, lambda b: (0,) * len(shape))
  nb = x.shape[0]
  return pl.pallas_call(
      _attn_body,
      grid=(nb,),
      in_specs=[
          pl.BlockSpec(memory_space=pltpu.SMEM),
          pl.BlockSpec((1, L, D), lambda b: (b, 0, 0)),
          full((D, D)), full((D, D)), full((D, D)), full((D, D)),
          full((1, D)), full((1, D)), full((1, D)), full((1, D)),
          full((1, D)), full((1, D)),
      ],
      out_specs=pl.BlockSpec((1, L, D), lambda b: (b, 0, 0)),
      out_shape=jax.ShapeDtypeStruct((nb, L, D), jnp.float32),
      scratch_shapes=[pltpu.VMEM((L, D), jnp.float32)],
  )(lengths, x, Wq, Wk, Wv, Wo, bq, bk, bv, bo, ln1_g, ln1_b)


_FR = 128
_FSTEPS = (B * L) // _FR
_WNB = N // _FSTEPS
_WPC = 1024


def _ffn_body(len_ref, h_ref, w1_ref, w2_ref, b1_ref, b2_ref, g2_ref, t2_ref,
              out_ref):
  i = pl.program_id(0)
  nsub = L // _FR
  e = i // nsub
  j = i % nsub

  @pl.when(j * _FR < len_ref[e, 0])
  def _():
    h1 = h_ref[...]
    mid = jnp.maximum(
        jnp.dot(h1, w1_ref[...], preferred_element_type=jnp.float32)
        + b1_ref[...], 0.0)
    ff = (jnp.dot(mid, w2_ref[...], preferred_element_type=jnp.float32)
          + b2_ref[...])
    out_ref[...] = _ln_rows(h1 + ff, g2_ref[...], t2_ref[...])


def _ffn(h1, lengths, W1, W2, b1, b2, ln2_g, ln2_b):
  full = lambda shape: pl.BlockSpec(shape, lambda i: (0,) * len(shape))
  rows = B * L
  return pl.pallas_call(
      _ffn_body,
      grid=(rows // _FR,),
      in_specs=[
          pl.BlockSpec(memory_space=pltpu.SMEM),
          pl.BlockSpec((_FR, D), lambda i: (i, 0)),
          full((D, DFF)), full((DFF, D)),
          full((1, DFF)), full((1, D)), full((1, D)), full((1, D)),
      ],
      out_specs=pl.BlockSpec((_FR, D), lambda i: (i, 0)),
      out_shape=jax.ShapeDtypeStruct((rows, D), jnp.float32),
  )(lengths, h1, W1, W2, b1, b2, ln2_g, ln2_b)


_NB = 8
_NBS = N // _NB
_PBS = 1024


def _winner_body(idx_ref, widx_ref, valid_ref):
  for nb in range(_NB):
    nids = nb * _NBS + lax.broadcasted_iota(jnp.int32, (1, _NBS), 1)
    best = jnp.full((1, _NBS), -1, jnp.int32)
    for pb in range(B * L // _PBS):
      c = idx_ref[pl.ds(pb * _PBS, _PBS), :]
      pos = pb * _PBS + lax.broadcasted_iota(jnp.int32, (_PBS, _NBS), 0)
      cand = jnp.where(c == nids, pos, -1)
      best = jnp.maximum(best, jnp.max(cand, axis=0, keepdims=True))
    widx_ref[pl.ds(nb, 1), :] = jnp.maximum(best, 0)
    valid_ref[pl.ds(nb, 1), :] = (best >= 0).astype(jnp.float32)


def _winner(idx_flat):
  return pl.pallas_call(
      _winner_body,
      in_specs=[pl.BlockSpec((B * L, 1), lambda: (0, 0))],
      out_specs=[pl.BlockSpec((_NB, _NBS), lambda: (0, 0)),
                 pl.BlockSpec((_NB, _NBS), lambda: (0, 0))],
      out_shape=[jax.ShapeDtypeStruct((_NB, _NBS), jnp.int32),
                 jax.ShapeDtypeStruct((_NB, _NBS), jnp.float32)],
  )(idx_flat.reshape(B * L, 1))


def _final_ln_body(y_ref, valid_ref, g_ref, b_ref, out_ref):
  y = y_ref[...] * valid_ref[...]
  out_ref[...] = _ln_rows(y, g_ref[...], b_ref[...])


def _final_ln(rows, valid, norm_g, norm_b):
  blk = 512
  return pl.pallas_call(
      _final_ln_body,
      grid=(N // blk,),
      in_specs=[
          pl.BlockSpec((blk, D), lambda i: (i, 0)),
          pl.BlockSpec((blk, 1), lambda i: (i, 0)),
          pl.BlockSpec((1, D), lambda i: (0, 0)),
          pl.BlockSpec((1, D), lambda i: (0, 0)),
      ],
      out_specs=pl.BlockSpec((blk, D), lambda i: (i, 0)),
      out_shape=jax.ShapeDtypeStruct((N, D), jnp.float32),
  )(rows, valid, norm_g, norm_b)


def kernel(cfg_nodes_encodings, permutations, lengths, Wq, bq, Wk, bk, Wv, bv,
           Wo, bo, W1, b1, W2, b2, ln1_g, ln1_b, ln2_g, ln2_b, norm_g, norm_b):
  perm_flat = permutations.reshape(B * L)
  x_flat = _sc_gather(cfg_nodes_encodings, perm_flat, B * L, 64)

  mask = jnp.arange(L, dtype=jnp.int32)[None, :] < lengths[:, None]
  idx_flat = jnp.where(mask, permutations, N).reshape(B * L)
  widx, valid = _winner(idx_flat)

  lens2 = lengths.reshape(B, 1)
  h1 = _attention(x_flat.reshape(B, L, D), lens2, Wq, Wk, Wv, Wo,
                  bq.reshape(1, D), bk.reshape(1, D), bv.reshape(1, D),
                  bo.reshape(1, D), ln1_g.reshape(1, D), ln1_b.reshape(1, D))

  h = _ffn(h1.reshape(B * L, D), lens2, W1, W2,
           b1.reshape(1, DFF), b2.reshape(1, D),
           ln2_g.reshape(1, D), ln2_b.reshape(1, D))

  rows = _sc_gather(h, widx.reshape(N), N, 64)
  return _final_ln(rows, valid.reshape(N, 1),
                   norm_g.reshape(1, D), norm_b.reshape(1, D))
